# Initial kernel scaffold; baseline (speedup 1.0000x reference)
import jax
import jax.numpy as jnp
from jax.experimental import pallas as pl


def kernel(x, emb_table, proj_w, proj_b):
    raise NotImplementedError("write your pallas kernel here")


# speedup vs baseline: 2.9498x; 2.9498x over previous
import functools

import jax
import jax.numpy as jnp
from jax import lax
from jax.experimental import pallas as pl
from jax.experimental.pallas import tpu as pltpu
from jax.experimental.pallas import tpu_sc as plsc

D = 1024
KGRP = 4
N_TOKENS = 8192
N_ROWS = N_TOKENS * KGRP
NW = 32
ROWS_PER_W = N_ROWS // NW
CHUNK = 64
NCHUNK = ROWS_PER_W // CHUNK


def _sc_gather(table, idx):
    mesh = plsc.VectorSubcoreMesh(core_axis_name="c", subcore_axis_name="s")

    @functools.partial(
        pl.kernel,
        mesh=mesh,
        out_type=jax.ShapeDtypeStruct((N_ROWS, D), jnp.float32),
        scratch_types=[
            pltpu.VMEM((CHUNK,), jnp.int32),
            pltpu.VMEM((CHUNK, D), jnp.float32),
            pltpu.SemaphoreType.DMA,
        ],
    )
    def gather_kernel(table_hbm, idx_hbm, out_hbm, idx_v, rows_v, sem):
        wid = lax.axis_index("s") * 2 + lax.axis_index("c")
        base = wid * ROWS_PER_W

        def body(i, carry):
            rb = base + i * CHUNK
            pltpu.sync_copy(idx_hbm.at[pl.ds(rb, CHUNK)], idx_v)
            pltpu.async_copy(table_hbm.at[idx_v], rows_v, sem).wait()
            pltpu.sync_copy(rows_v, out_hbm.at[pl.ds(rb, CHUNK)])
            return carry

        lax.fori_loop(0, NCHUNK, body, 0)

    return gather_kernel(table, idx)


_TM = 512


def _tc_matmul(flat, w, b2d):
    def body(a_ref, w_ref, b_ref, o_ref):
        a = a_ref[...].astype(jnp.bfloat16)
        wt = w_ref[...].astype(jnp.bfloat16)
        acc = lax.dot_general(
            a, wt, (((1,), (1,)), ((), ())), preferred_element_type=jnp.float32
        )
        o_ref[...] = acc + b_ref[...]

    return pl.pallas_call(
        body,
        grid=(N_TOKENS // _TM,),
        in_specs=[
            pl.BlockSpec((_TM, KGRP * D), lambda i: (i, 0)),
            pl.BlockSpec((D, KGRP * D), lambda i: (0, 0)),
            pl.BlockSpec((1, D), lambda i: (0, 0)),
        ],
        out_specs=pl.BlockSpec((_TM, D), lambda i: (i, 0)),
        out_shape=jax.ShapeDtypeStruct((N_TOKENS, D), jnp.float32),
    )(flat, w, b2d)


def kernel(x, emb_table, proj_w, proj_b):
    B, L, K = x.shape
    idx = x.reshape(-1).astype(jnp.int32)
    flat = _sc_gather(emb_table, idx)
    flat2 = flat.reshape(N_TOKENS, KGRP * D)
    out = _tc_matmul(flat2, proj_w, proj_b.reshape(1, D))
    return out.reshape(B, L, D)


# speedup vs baseline: 5.1379x; 1.7418x over previous
import functools

import jax
import jax.numpy as jnp
from jax import lax
from jax.experimental import pallas as pl
from jax.experimental.pallas import tpu as pltpu
from jax.experimental.pallas import tpu_sc as plsc

D = 1024
KGRP = 4
N_TOKENS = 8192
N_ROWS = N_TOKENS * KGRP
NW = 32
ROWS_PER_W = N_ROWS // NW
CHUNK = 64
NCHUNK = ROWS_PER_W // CHUNK


def _sc_gather(table, idx):
    mesh = plsc.VectorSubcoreMesh(core_axis_name="c", subcore_axis_name="s")

    @functools.partial(
        pl.kernel,
        mesh=mesh,
        out_type=jax.ShapeDtypeStruct((N_ROWS, D), jnp.float32),
        scratch_types=[
            pltpu.VMEM((CHUNK,), jnp.int32),
            pltpu.VMEM((CHUNK, D), jnp.float32),
            pltpu.SemaphoreType.DMA,
        ],
    )
    def gather_kernel(table_hbm, idx_hbm, out_hbm, idx_v, rows_v, sem):
        wid = lax.axis_index("s") * 2 + lax.axis_index("c")
        base = wid * ROWS_PER_W

        def body(i, carry):
            rb = base + i * CHUNK
            pltpu.sync_copy(idx_hbm.at[pl.ds(rb, CHUNK)], idx_v)
            pltpu.async_copy(table_hbm.at[idx_v], rows_v, sem).wait()
            pltpu.sync_copy(rows_v, out_hbm.at[pl.ds(rb, CHUNK)])
            return carry

        lax.fori_loop(0, NCHUNK, body, 0)

    return gather_kernel(table, idx)


_TM = 512


def _tc_matmul(planes, w, b2d):
    def body(a_ref, w_ref, b_ref, o_ref):
        acc = b_ref[...].astype(jnp.float32)
        acc = jnp.broadcast_to(acc, (_TM, D))
        for k in range(KGRP):
            a = a_ref[k].astype(jnp.bfloat16)
            wk = w_ref[:, k * D:(k + 1) * D].astype(jnp.bfloat16)
            acc = acc + lax.dot_general(
                a, wk, (((1,), (1,)), ((), ())),
                preferred_element_type=jnp.float32,
            )
        o_ref[...] = acc

    return pl.pallas_call(
        body,
        grid=(N_TOKENS // _TM,),
        in_specs=[
            pl.BlockSpec((KGRP, _TM, D), lambda i: (0, i, 0)),
            pl.BlockSpec((D, KGRP * D), lambda i: (0, 0)),
            pl.BlockSpec((1, D), lambda i: (0, 0)),
        ],
        out_specs=pl.BlockSpec((_TM, D), lambda i: (i, 0)),
        out_shape=jax.ShapeDtypeStruct((N_TOKENS, D), jnp.float32),
    )(planes, w, b2d)


def kernel(x, emb_table, proj_w, proj_b):
    B, L, K = x.shape
    idx = x.reshape(-1, K).T.reshape(-1).astype(jnp.int32)
    flat = _sc_gather(emb_table, idx)
    planes = flat.reshape(KGRP, N_TOKENS, D)
    out = _tc_matmul(planes, proj_w, proj_b.reshape(1, D))
    return out.reshape(B, L, D)
